# speedup vs baseline: 8.0167x; 8.0167x over previous
import functools

import jax
import jax.numpy as jnp
from jax.experimental import pallas as pl

_B, _C, _H, _W = 8, 768, 32, 32
_SCALES = (
    (4, 29, 6, 0, 0.25),
    (6, 27, 5, 841, 0.25),
    (8, 25, 4, 1570, 0.25),
)
_TOTAL = 2195
_NUM_PROPOSALS = 15


def _pool(fm, r):
    side = _H - r + 1
    hs = fm[:, 0:side]
    for d in range(1, r):
        hs = hs + fm[:, d:d + side]
    ps = hs[0:side, :]
    for d in range(1, r):
        ps = ps + hs[d:d + side, :]
    return ps * (1.0 / float(r * r))


def _nms_scale(pooled, r, side, nsel, base, sel_idx, sel_score):
    rowi = jax.lax.broadcasted_iota(jnp.int32, (side, side), 0)
    colj = jax.lax.broadcasted_iota(jnp.int32, (side, side), 1)
    flat = rowi * side + colj
    two_area = jnp.int32(2 * 256 * r * r)
    supp = jnp.zeros((side, side), dtype=jnp.bool_)
    for _ in range(nsel):
        masked = jnp.where(supp, -jnp.inf, pooled)
        m = jnp.max(masked)
        idxflat = jnp.min(jnp.where(masked == m, flat, jnp.int32(10**6)))
        i0 = idxflat // side
        j0 = idxflat - i0 * side
        u = jnp.maximum(0, r - jnp.abs(rowi - i0))
        v = jnp.maximum(0, r - jnp.abs(colj - j0))
        inter = u * v * 256
        supp = supp | (5 * inter > two_area) | (flat == idxflat)
        sel_idx.append((idxflat + base).astype(jnp.float32))
        sel_score.append(m)


def _kernel_body(x_ref, ws4_ref, ws6_ref, ws8_ref, idx_ref, scr_ref):
    fm = jnp.sum(x_ref[0], axis=0)
    sel_idx = []
    sel_score = []
    ws_refs = (ws4_ref, ws6_ref, ws8_ref)
    for (r, side, nsel, base, _), ws_ref in zip(_SCALES, ws_refs):
        pooled = _pool(fm, r)
        ws_ref[0] = pooled
        _nms_scale(pooled, r, side, nsel, base, sel_idx, sel_score)
    idx_ref[0, 0, :] = jnp.stack(sel_idx)
    scr_ref[0, 0, :] = jnp.stack(sel_score)


@jax.jit
def _run(input_tensor):
    out_types = (
        jax.ShapeDtypeStruct((_B, 29, 29), jnp.float32),
        jax.ShapeDtypeStruct((_B, 27, 27), jnp.float32),
        jax.ShapeDtypeStruct((_B, 25, 25), jnp.float32),
        jax.ShapeDtypeStruct((_B, 1, _NUM_PROPOSALS), jnp.float32),
        jax.ShapeDtypeStruct((_B, 1, _NUM_PROPOSALS), jnp.float32),
    )
    ws4, ws6, ws8, idx_f, scr = pl.pallas_call(
        _kernel_body,
        grid=(_B,),
        in_specs=[pl.BlockSpec((1, _C, _H, _W), lambda b: (b, 0, 0, 0))],
        out_specs=(
            pl.BlockSpec((1, 29, 29), lambda b: (b, 0, 0)),
            pl.BlockSpec((1, 27, 27), lambda b: (b, 0, 0)),
            pl.BlockSpec((1, 25, 25), lambda b: (b, 0, 0)),
            pl.BlockSpec((1, 1, _NUM_PROPOSALS), lambda b: (b, 0, 0)),
            pl.BlockSpec((1, 1, _NUM_PROPOSALS), lambda b: (b, 0, 0)),
        ),
        out_shape=out_types,
    )(input_tensor)
    window_scores = jnp.concatenate(
        [ws4.reshape(_B, -1), ws6.reshape(_B, -1), ws8.reshape(_B, -1)], axis=1)
    proposal_indices = idx_f.reshape(_B, _NUM_PROPOSALS).astype(jnp.int32)
    proposal_scores = scr.reshape(_B, _NUM_PROPOSALS)
    return proposal_indices, proposal_scores, window_scores


def kernel(input_tensor, coordinates_cat, num_proposals, pooling_ratios,
           window_nums_sum, N_list, iou_thresholds):
    return _run(input_tensor)


# speedup vs baseline: 15.9802x; 1.9934x over previous
import functools

import jax
import jax.numpy as jnp
from jax import lax
from jax.experimental import pallas as pl
from jax.experimental.pallas import tpu as pltpu
from jax.experimental.pallas import tpu_sc as plsc

_B, _C, _H, _W = 8, 768, 32, 32
_HW = _H * _W
_SCALES = (
    (4, 29, 6, 0),
    (6, 27, 5, 841),
    (8, 25, 4, 1570),
)
_NUM_PROPOSALS = 15
_NEG_INF = float("-inf")


def _lane_reduce(vec, op):
    vals = [vec[i] for i in range(16)]
    while len(vals) > 1:
        vals = [op(vals[i], vals[i + 1]) for i in range(0, len(vals), 2)]
    return vals[0]


def _pool_1d(fm, r):
    acc = {1: fm}

    def widen(a_w, w, b_v, v):
        return a_w + jnp.roll(b_v, -w, axis=1)

    acc[2] = widen(acc[1], 1, acc[1], 1)
    acc[4] = widen(acc[2], 2, acc[2], 2)
    if r == 4:
        hs = acc[4]
    elif r == 6:
        hs = widen(acc[4], 4, acc[2], 2)
    else:
        hs = widen(acc[4], 4, acc[4], 4)
    vcc = {1: hs}
    vcc[2] = vcc[1] + jnp.roll(vcc[1], -32, axis=1)
    vcc[4] = vcc[2] + jnp.roll(vcc[2], -64, axis=1)
    if r == 4:
        ps = vcc[4]
    elif r == 6:
        ps = vcc[4] + jnp.roll(vcc[2], -128, axis=1)
    else:
        ps = vcc[4] + jnp.roll(vcc[4], -128, axis=1)
    return ps * (1.0 / float(r * r))


def _tc_body(x_ref, out_ref):
    fm = jnp.dot(jnp.ones((1, _C), jnp.float32), x_ref[0],
                 preferred_element_type=jnp.float32,
                 precision=jax.lax.Precision.HIGHEST)
    for j, (r, _, _, _) in enumerate(_SCALES):
        out_ref[0, j] = _pool_1d(fm, r)[0]


@jax.jit
def _tc_scores(x):
    return pl.pallas_call(
        _tc_body,
        grid=(_B,),
        in_specs=[pl.BlockSpec((1, _C, _HW), lambda b: (b, 0, 0))],
        out_specs=pl.BlockSpec((1, 3, _HW), lambda b: (b, 0, 0)),
        out_shape=jax.ShapeDtypeStruct((_B, 3, _HW), jnp.float32),
    )(x)


def _sc_nms_scale(r, side, nsel, base, b, s_ref, mask_ref, idxv_ref, scrv_ref):
    iota = lax.broadcasted_iota(jnp.int32, (16,), 0)

    def init_row(i, _):
        for h in range(2):
            mask_ref[pl.ds(i * 32 + h * 16, 16)] = jnp.where(
                h * 16 + iota < side, 0.0, _NEG_INF)
        return 0

    lax.fori_loop(0, side, init_row, 0)

    out_idx = jnp.zeros((16,), jnp.int32)
    out_scr = jnp.zeros((16,), jnp.float32)
    for k in range(nsel):
        def max_row(i, vmax):
            for h in range(2):
                d = pl.ds(i * 32 + h * 16, 16)
                vmax = jnp.maximum(vmax, s_ref[d] + mask_ref[d])
            return vmax

        m = _lane_reduce(
            lax.fori_loop(0, side, max_row,
                          jnp.full((16,), _NEG_INF, jnp.float32)),
            jnp.maximum)

        def arg_row(i, vmin):
            for h in range(2):
                d = pl.ds(i * 32 + h * 16, 16)
                p = i * 32 + h * 16 + iota
                cand = jnp.where(s_ref[d] + mask_ref[d] == m, p,
                                 jnp.int32(2**30))
                vmin = jnp.minimum(vmin, cand)
            return vmin

        p32 = _lane_reduce(
            lax.fori_loop(0, side, arg_row,
                          jnp.full((16,), 2**30, jnp.int32)),
            jnp.minimum)
        i0 = lax.shift_right_logical(p32, 5)
        j0 = lax.bitwise_and(p32, 31)

        def supp_row(ii, _):
            u = r - jnp.abs(ii - i0)
            for h in range(2):
                pj = h * 16 + iota
                v = jnp.maximum(0, r - jnp.abs(pj - j0))
                cond = 5 * u * v > 2 * r * r
                d = pl.ds(ii * 32 + h * 16, 16)
                mask_ref[d] = jnp.where(cond, _NEG_INF, mask_ref[d])
            return 0

        lax.fori_loop(jnp.maximum(0, i0 - (r - 1)),
                      jnp.minimum(side, i0 + r), supp_row, 0)

        gidx = i0 * side + j0 + base
        out_idx = jnp.where(iota == k, gidx, out_idx)
        out_scr = jnp.where(iota == k, m, out_scr)

    idxv_ref[...] = out_idx
    scrv_ref[...] = out_scr


def _sc_nms_kernel():
    info = plsc.get_sparse_core_info()
    nc = info.num_cores

    @functools.partial(
        pl.kernel,
        mesh=plsc.VectorSubcoreMesh(core_axis_name="c", subcore_axis_name="s"),
        out_type=(
            jax.ShapeDtypeStruct((_B, 3, 16), jnp.int32),
            jax.ShapeDtypeStruct((_B, 3, 16), jnp.float32),
        ),
        scratch_types=[
            pltpu.VMEM((_HW,), jnp.float32),
            pltpu.VMEM((_HW,), jnp.float32),
            pltpu.VMEM((16,), jnp.int32),
            pltpu.VMEM((16,), jnp.float32),
        ],
    )
    def nms(scores_hbm, idx_hbm, scr_hbm, s_v, mask_v, idxv, scrv):
        wid = lax.axis_index("s") * nc + lax.axis_index("c")
        b = wid % _B
        j = wid // _B

        @pl.when(wid < _B * 3)
        def _():
            pltpu.sync_copy(scores_hbm.at[b, j], s_v)
            for jj, (r, side, nsel, base) in enumerate(_SCALES):
                @pl.when(j == jj)
                def _():
                    _sc_nms_scale(r, side, nsel, base, b,
                                  s_v, mask_v, idxv, scrv)
            pltpu.sync_copy(idxv, idx_hbm.at[b, j])
            pltpu.sync_copy(scrv, scr_hbm.at[b, j])

    return nms


@jax.jit
def _run(input_tensor):
    packed = _tc_scores(input_tensor.reshape(_B, _C, _HW))
    idx_p, scr_p = _sc_nms_kernel()(packed)
    maps = packed.reshape(_B, 3, _H, _W)
    window_scores = jnp.concatenate(
        [maps[:, jj, :side, :side].reshape(_B, side * side)
         for jj, (_, side, _, _) in enumerate(_SCALES)], axis=1)
    proposal_indices = jnp.concatenate(
        [idx_p[:, jj, :nsel] for jj, (_, _, nsel, _) in enumerate(_SCALES)],
        axis=1)
    proposal_scores = jnp.concatenate(
        [scr_p[:, jj, :nsel] for jj, (_, _, nsel, _) in enumerate(_SCALES)],
        axis=1)
    return proposal_indices, proposal_scores, window_scores


def kernel(input_tensor, coordinates_cat, num_proposals, pooling_ratios,
           window_nums_sum, N_list, iou_thresholds):
    return _run(input_tensor)


# speedup vs baseline: 17.3153x; 1.0835x over previous
import functools

import jax
import jax.numpy as jnp
from jax import lax
from jax.experimental import pallas as pl
from jax.experimental.pallas import tpu as pltpu
from jax.experimental.pallas import tpu_sc as plsc

_B, _C, _H, _W = 8, 768, 32, 32
_HW = _H * _W
_SCALES = (
    (4, 29, 6, 0),
    (6, 27, 5, 841),
    (8, 25, 4, 1570),
)
_NUM_PROPOSALS = 15
_NEG_INF = float("-inf")


def _lane_reduce(vec, op):
    vals = [vec[i] for i in range(16)]
    while len(vals) > 1:
        vals = [op(vals[i], vals[i + 1]) for i in range(0, len(vals), 2)]
    return vals[0]


def _pool_1d(fm, r):
    acc = {1: fm}

    def widen(a_w, w, b_v, v):
        return a_w + jnp.roll(b_v, -w, axis=1)

    acc[2] = widen(acc[1], 1, acc[1], 1)
    acc[4] = widen(acc[2], 2, acc[2], 2)
    if r == 4:
        hs = acc[4]
    elif r == 6:
        hs = widen(acc[4], 4, acc[2], 2)
    else:
        hs = widen(acc[4], 4, acc[4], 4)
    vcc = {1: hs}
    vcc[2] = vcc[1] + jnp.roll(vcc[1], -32, axis=1)
    vcc[4] = vcc[2] + jnp.roll(vcc[2], -64, axis=1)
    if r == 4:
        ps = vcc[4]
    elif r == 6:
        ps = vcc[4] + jnp.roll(vcc[2], -128, axis=1)
    else:
        ps = vcc[4] + jnp.roll(vcc[4], -128, axis=1)
    return ps * (1.0 / float(r * r))


def _tc_body(x_ref, out_ref):
    fm = jnp.sum(x_ref[0], axis=0, keepdims=True)
    for j, (r, _, _, _) in enumerate(_SCALES):
        out_ref[0, j] = _pool_1d(fm, r)[0]


@jax.jit
def _tc_scores(x):
    return pl.pallas_call(
        _tc_body,
        grid=(_B,),
        in_specs=[pl.BlockSpec((1, _C, _HW), lambda b: (b, 0, 0))],
        out_specs=pl.BlockSpec((1, 3, _HW), lambda b: (b, 0, 0)),
        out_shape=jax.ShapeDtypeStruct((_B, 3, _HW), jnp.float32),
    )(x)


def _sc_nms_scale(r, side, nsel, base, b, s_ref, mask_ref, idxv_ref, scrv_ref):
    iota = lax.broadcasted_iota(jnp.int32, (16,), 0)

    def init_row(i, _):
        for h in range(2):
            mask_ref[pl.ds(i * 32 + h * 16, 16)] = jnp.where(
                h * 16 + iota < side, 0.0, _NEG_INF)
        return 0

    lax.fori_loop(0, side, init_row, 0)

    out_idx = jnp.zeros((16,), jnp.int32)
    out_scr = jnp.zeros((16,), jnp.float32)
    for k in range(nsel):
        def max_row(i, vmax):
            for h in range(2):
                d = pl.ds(i * 32 + h * 16, 16)
                vmax = jnp.maximum(vmax, s_ref[d] + mask_ref[d])
            return vmax

        m = _lane_reduce(
            lax.fori_loop(0, side, max_row,
                          jnp.full((16,), _NEG_INF, jnp.float32)),
            jnp.maximum)

        def arg_row(i, vmin):
            for h in range(2):
                d = pl.ds(i * 32 + h * 16, 16)
                p = i * 32 + h * 16 + iota
                cand = jnp.where(s_ref[d] + mask_ref[d] == m, p,
                                 jnp.int32(2**30))
                vmin = jnp.minimum(vmin, cand)
            return vmin

        p32 = _lane_reduce(
            lax.fori_loop(0, side, arg_row,
                          jnp.full((16,), 2**30, jnp.int32)),
            jnp.minimum)
        i0 = lax.shift_right_logical(p32, 5)
        j0 = lax.bitwise_and(p32, 31)

        def supp_row(ii, _):
            u = r - jnp.abs(ii - i0)
            for h in range(2):
                pj = h * 16 + iota
                v = jnp.maximum(0, r - jnp.abs(pj - j0))
                cond = 5 * u * v > 2 * r * r
                d = pl.ds(ii * 32 + h * 16, 16)
                mask_ref[d] = jnp.where(cond, _NEG_INF, mask_ref[d])
            return 0

        lax.fori_loop(jnp.maximum(0, i0 - (r - 1)),
                      jnp.minimum(side, i0 + r), supp_row, 0)

        gidx = i0 * side + j0 + base
        out_idx = jnp.where(iota == k, gidx, out_idx)
        out_scr = jnp.where(iota == k, m, out_scr)

    idxv_ref[...] = out_idx
    scrv_ref[...] = out_scr


def _sc_nms_kernel():
    info = plsc.get_sparse_core_info()
    nc = info.num_cores

    @functools.partial(
        pl.kernel,
        mesh=plsc.VectorSubcoreMesh(core_axis_name="c", subcore_axis_name="s"),
        out_type=(
            jax.ShapeDtypeStruct((_B, 3, 16), jnp.int32),
            jax.ShapeDtypeStruct((_B, 3, 16), jnp.float32),
        ),
        scratch_types=[
            pltpu.VMEM((_HW,), jnp.float32),
            pltpu.VMEM((_HW,), jnp.float32),
            pltpu.VMEM((16,), jnp.int32),
            pltpu.VMEM((16,), jnp.float32),
        ],
    )
    def nms(scores_hbm, idx_hbm, scr_hbm, s_v, mask_v, idxv, scrv):
        wid = lax.axis_index("s") * nc + lax.axis_index("c")
        b = wid % _B
        j = wid // _B

        @pl.when(wid < _B * 3)
        def _():
            pltpu.sync_copy(scores_hbm.at[b, j], s_v)
            for jj, (r, side, nsel, base) in enumerate(_SCALES):
                @pl.when(j == jj)
                def _():
                    _sc_nms_scale(r, side, nsel, base, b,
                                  s_v, mask_v, idxv, scrv)
            pltpu.sync_copy(idxv, idx_hbm.at[b, j])
            pltpu.sync_copy(scrv, scr_hbm.at[b, j])

    return nms


@jax.jit
def _run(input_tensor):
    packed = _tc_scores(input_tensor.reshape(_B, _C, _HW))
    idx_p, scr_p = _sc_nms_kernel()(packed)
    maps = packed.reshape(_B, 3, _H, _W)
    window_scores = jnp.concatenate(
        [maps[:, jj, :side, :side].reshape(_B, side * side)
         for jj, (_, side, _, _) in enumerate(_SCALES)], axis=1)
    proposal_indices = jnp.concatenate(
        [idx_p[:, jj, :nsel] for jj, (_, _, nsel, _) in enumerate(_SCALES)],
        axis=1)
    proposal_scores = jnp.concatenate(
        [scr_p[:, jj, :nsel] for jj, (_, _, nsel, _) in enumerate(_SCALES)],
        axis=1)
    return proposal_indices, proposal_scores, window_scores


def kernel(input_tensor, coordinates_cat, num_proposals, pooling_ratios,
           window_nums_sum, N_list, iou_thresholds):
    return _run(input_tensor)
